# baseline (speedup 1.0000x reference)
import functools

import jax
import jax.numpy as jnp
from jax import lax
from jax.experimental import pallas as pl
from jax.experimental.pallas import tpu as pltpu
from jax.experimental.pallas import tpu_sc as plsc

B = 128
N = 32768
KTOP = 8
L = 16
NCHUNK = N // L
NSTRIP = 16
CPS = NCHUNK // NSTRIP
SUBC = 16
NSUB = CPS // SUBC
UN = 16
NW = 32
ROWS_PER = B // NW

NEG = float("-inf")
BIG = 1 << 30


def _topk_rows(w_hbm, out_hbm, rowbuf0, rowbuf1, outbuf, subs, sem_in, sem_out):
    cid = lax.axis_index("c")
    sid = lax.axis_index("s")
    wid = sid * 2 + cid
    lanes = lax.iota(jnp.int32, L)
    zeros16 = jnp.zeros((L,), jnp.float32)
    neg16 = jnp.full((L,), NEG, jnp.float32)
    big16 = jnp.full((L,), BIG, jnp.int32)
    zi16 = jnp.zeros((L,), jnp.int32)
    lane0 = lanes == 0
    sel8 = lanes < KTOP

    bufs = (rowbuf0, rowbuf1)
    base_row = wid * ROWS_PER
    h_in = pltpu.async_copy(w_hbm.at[base_row], rowbuf0, sem_in)

    def zero_body(i, c):
        for u in range(UN):
            outbuf[pl.ds((i * UN + u) * L, L)] = zeros16
        return c
    lax.fori_loop(0, NCHUNK // UN, zero_body, 0)

    h_out = None
    idx_prev = None
    for r in range(ROWS_PER):
        rb = bufs[r % 2]
        h_in.wait()
        if r + 1 < ROWS_PER:
            h_in = pltpu.async_copy(
                w_hbm.at[base_row + r + 1], bufs[(r + 1) % 2], sem_in)

        smax16 = neg16
        for j in range(NSTRIP):
            def smax_body(s, mx, _j=j, _rb=rb):
                base = (_j * CPS + s * SUBC) * L
                msub = _rb[pl.ds(base, L)]
                for u in range(1, SUBC):
                    msub = jnp.maximum(msub, _rb[pl.ds(base + u * L, L)])
                subs[pl.ds((_j * NSUB + s) * L, L)] = msub
                return jnp.maximum(mx, msub)
            mx = lax.fori_loop(0, NSUB, smax_body, neg16)
            smax16 = jnp.where(lanes == j, jnp.max(mx), smax16)

        vals8 = zeros16
        idx8 = zi16
        for it in range(KTOP):
            gmax = jnp.max(smax16)
            minj = jnp.min(jnp.where(smax16 == gmax, lanes, jnp.int32(99)))

            ms = jnp.full((L,), 99, jnp.int32)
            for s in range(NSUB):
                sub = subs[pl.ds((minj * NSUB + s) * L, L)]
                ms = jnp.minimum(ms, jnp.where(sub == gmax, jnp.int32(s), jnp.int32(99)))
            mins = jnp.min(ms)

            sbase = (minj * CPS + mins * SUBC) * L
            midx, cnt, mlt = big16, zi16, neg16
            for u in range(SUBC):
                v = rb[pl.ds(sbase + u * L, L)]
                eq = v == gmax
                midx = jnp.minimum(midx, jnp.where(eq, sbase + u * L + lanes, BIG))
                cnt = cnt + eq.astype(jnp.int32)
                mlt = jnp.maximum(mlt, jnp.where(eq, NEG, v))
            idx = jnp.min(midx)

            plsc.store_scatter(rb, [jnp.full((L,), idx)], neg16, mask=lane0)
            cnt_adj = cnt - (lanes == (idx & (L - 1))).astype(jnp.int32)
            newslice = jnp.where(cnt_adj > 0, gmax, mlt)
            subs[pl.ds((minj * NSUB + mins) * L, L)] = newslice
            mstrip = newslice
            for s in range(NSUB):
                sub = subs[pl.ds((minj * NSUB + s) * L, L)]
                mstrip = jnp.maximum(mstrip, jnp.where(jnp.int32(s) == mins, neg16, sub))
            smax16 = jnp.where(lanes == minj, jnp.max(mstrip), smax16)

            vals8 = jnp.where(lanes == it, gmax, vals8)
            idx8 = jnp.where(lanes == it, idx, idx8)

        l1 = jnp.sum(jnp.where(sel8, jnp.abs(vals8), 0.0))
        invv = 1.0 / jnp.maximum(jnp.full((L,), l1), jnp.float32(1e-12))

        if r > 0:
            h_out.wait()
            plsc.store_scatter(outbuf, [idx_prev], zeros16, mask=sel8)
        plsc.store_scatter(outbuf, [idx8], vals8 * invv, mask=sel8)
        h_out = pltpu.async_copy(outbuf, out_hbm.at[base_row + r], sem_out)
        idx_prev = idx8
    h_out.wait()


def kernel(weights, k):
    del k
    mesh = plsc.VectorSubcoreMesh(core_axis_name="c", subcore_axis_name="s")
    run = functools.partial(
        pl.kernel,
        mesh=mesh,
        compiler_params=pltpu.CompilerParams(needs_layout_passes=False),
        out_type=jax.ShapeDtypeStruct((B, N), jnp.float32),
        scratch_types=[
            pltpu.VMEM((N,), jnp.float32),
            pltpu.VMEM((N,), jnp.float32),
            pltpu.VMEM((N,), jnp.float32),
            pltpu.VMEM((NSTRIP * NSUB * L,), jnp.float32),
            pltpu.SemaphoreType.DMA,
            pltpu.SemaphoreType.DMA,
        ],
    )(_topk_rows)
    return run(weights)
